# Initial kernel scaffold; baseline (speedup 1.0000x reference)
import jax
import jax.numpy as jnp
from jax.experimental import pallas as pl


def kernel(inputs):
    raise NotImplementedError("write your pallas kernel here")


# speedup vs baseline: 6.6688x; 6.6688x over previous
import jax
import jax.numpy as jnp
from jax.experimental import pallas as pl

N_KEEP = 48 * 256
BATCH = 1024
ROWS_PER_BLOCK = 256


def _copy_body(in_ref, out_ref):
    out_ref[...] = in_ref[...]


def kernel(inputs):
    batch, _ = inputs.shape
    grid = (batch // ROWS_PER_BLOCK,)
    return pl.pallas_call(
        _copy_body,
        grid=grid,
        in_specs=[
            pl.BlockSpec((ROWS_PER_BLOCK, N_KEEP), lambda i: (i, 0)),
        ],
        out_specs=pl.BlockSpec((ROWS_PER_BLOCK, N_KEEP), lambda i: (i, 0)),
        out_shape=jax.ShapeDtypeStruct((batch, N_KEEP), inputs.dtype),
    )(inputs)
